# baseline (speedup 1.0000x reference)
import functools

import jax
import jax.numpy as jnp
from jax import lax
from jax.experimental import pallas as pl
from jax.experimental.pallas import tpu as pltpu
from jax.experimental.pallas import tpu_sc as plsc

_VOCAB = 100000
_EMBED = 64
_HIDDEN = 128
_CTX = 200

_NC = 2
_NS = 16
_NW = _NC * _NS
_IDX_PER_W = 8
_ACTIVE_W = _CTX // _IDX_PER_W

_TILE = 2048
_GRID = (_VOCAB + _TILE - 1) // _TILE
_EDGE = _VOCAB - (_GRID - 1) * _TILE


def _sc_gather_sum(idx_hbm, emb_hbm, out_hbm, idx_v, rows_v, acc_v, sem):
    wid = lax.axis_index("s") * _NC + lax.axis_index("c")

    @pl.when(wid < _ACTIVE_W)
    def _gather():
        base = pl.multiple_of(wid * _IDX_PER_W, _IDX_PER_W)
        pltpu.sync_copy(idx_hbm.at[pl.ds(base, _IDX_PER_W)], idx_v)
        pltpu.async_copy(emb_hbm.at[idx_v], rows_v, sem).wait()
        for c in range(_EMBED // 16):
            acc = rows_v[0, pl.ds(c * 16, 16)]
            for j in range(1, _IDX_PER_W):
                acc = acc + rows_v[j, pl.ds(c * 16, 16)]
            acc_v[0, pl.ds(c * 16, 16)] = acc

    @pl.when(wid >= _ACTIVE_W)
    def _zero():
        for c in range(_EMBED // 16):
            acc_v[0, pl.ds(c * 16, 16)] = jnp.zeros((16,), jnp.float32)

    pltpu.sync_copy(acc_v, out_hbm.at[pl.ds(wid, 1)])


@functools.cache
def _sc_gather():
    return pl.kernel(
        _sc_gather_sum,
        out_type=jax.ShapeDtypeStruct((_NW, _EMBED), jnp.float32),
        mesh=plsc.VectorSubcoreMesh(core_axis_name="c", subcore_axis_name="s"),
        scratch_types=[
            pltpu.VMEM((_IDX_PER_W,), jnp.int32),
            pltpu.VMEM((_IDX_PER_W, _EMBED), jnp.float32),
            pltpu.VMEM((1, _EMBED), jnp.float32),
            pltpu.SemaphoreType.DMA,
        ],
        compiler_params=pltpu.CompilerParams(use_tc_tiling_on_sc=False),
    )


def _tc_mlp_softmax(
    parts_ref, w1_ref, b1_ref, w2_ref, b2_ref,
    out_ref, hid_ref, logits_ref, lse_ref,
):
    p = pl.program_id(0)
    i = pl.program_id(1)

    @pl.when((p == 0) & (i == 0))
    def _head():
        x = jnp.sum(parts_ref[...], axis=0, keepdims=True)
        h = lax.dot_general(
            x, w1_ref[...], (((1,), (1,)), ((), ())),
            preferred_element_type=jnp.float32,
        ) + b1_ref[...]
        hid_ref[...] = jnp.maximum(h, 0.0)

    @pl.when(p == 0)
    def _logits():
        t = lax.dot_general(
            hid_ref[...], w2_ref[...], (((1,), (1,)), ((), ())),
            preferred_element_type=jnp.float32,
        ) + b2_ref[...].reshape(1, _TILE)
        nvalid = jnp.where(i == _GRID - 1, _EDGE, _TILE)
        t = jnp.where(
            lax.broadcasted_iota(jnp.int32, (1, _TILE), 1) < nvalid, t, -1e30
        )
        logits_ref[i] = t

    @pl.when((p == 1) & (i == 0))
    def _stats():
        full = logits_ref[...]
        m = jnp.max(full)
        lse_ref[0] = m + jnp.log(jnp.sum(jnp.exp(full - m)))

    @pl.when(p == 1)
    def _normalize():
        out_ref[...] = logits_ref[i] - lse_ref[0]


_tc_call = pl.pallas_call(
    _tc_mlp_softmax,
    grid=(2, _GRID),
    in_specs=[
        pl.BlockSpec((_NW, _EMBED), lambda p, i: (0, 0)),
        pl.BlockSpec((_HIDDEN, _EMBED), lambda p, i: (0, 0)),
        pl.BlockSpec((1, _HIDDEN), lambda p, i: (0, 0)),
        pl.BlockSpec((_TILE, _HIDDEN), lambda p, i: (i * (1 - p), 0)),
        pl.BlockSpec((_TILE,), lambda p, i: (i * (1 - p),)),
    ],
    out_specs=pl.BlockSpec((1, _TILE), lambda p, i: (0, i * p)),
    out_shape=jax.ShapeDtypeStruct((1, _VOCAB), jnp.float32),
    scratch_shapes=[
        pltpu.VMEM((1, _HIDDEN), jnp.float32),
        pltpu.VMEM((_GRID, 1, _TILE), jnp.float32),
        pltpu.SMEM((1,), jnp.float32),
    ],
    compiler_params=pltpu.CompilerParams(
        dimension_semantics=("arbitrary", "arbitrary"),
    ),
)


@jax.jit
def kernel(inputs, emb, W1, b1, W2, b2):
    parts = _sc_gather()(inputs, emb)
    return _tc_call(parts, W1, b1.reshape(1, _HIDDEN), W2, b2)


# speedup vs baseline: 1.4533x; 1.4533x over previous
import functools

import jax
import jax.numpy as jnp
from jax import lax
from jax.experimental import pallas as pl
from jax.experimental.pallas import tpu as pltpu
from jax.experimental.pallas import tpu_sc as plsc

_VOCAB = 100000
_EMBED = 64
_HIDDEN = 128
_CTX = 200

_NC = 2
_NS = 16
_NW = _NC * _NS
_IDX_PER_W = 8
_ACTIVE_W = _CTX // _IDX_PER_W

_TILE = 2048
_GRID = (_VOCAB + _TILE - 1) // _TILE
_EDGE = _VOCAB - (_GRID - 1) * _TILE


def _sc_gather_sum(idx_hbm, emb_hbm, out_hbm, idx_v, rows_v, acc_v, sem):
    wid = lax.axis_index("s") * _NC + lax.axis_index("c")

    @pl.when(wid < _ACTIVE_W)
    def _gather():
        base = pl.multiple_of(wid * _IDX_PER_W, _IDX_PER_W)
        pltpu.sync_copy(idx_hbm.at[pl.ds(base, _IDX_PER_W)],
                        idx_v.at[pl.ds(0, _IDX_PER_W)])
        iv = idx_v[...]
        lane = lax.iota(jnp.int32, 16)
        copies = []
        for j in range(_IDX_PER_W):
            row_j = jnp.sum(jnp.where(lane == j, iv, 0))
            copies.append(pltpu.make_async_copy(
                emb_hbm.at[pl.ds(row_j, 1)], rows_v.at[pl.ds(j, 1)], sem))
        for c in copies:
            c.start()
        for c in copies:
            c.wait()
        for c in range(_EMBED // 16):
            acc = rows_v[0, pl.ds(c * 16, 16)]
            for j in range(1, _IDX_PER_W):
                acc = acc + rows_v[j, pl.ds(c * 16, 16)]
            acc_v[0, pl.ds(c * 16, 16)] = acc

    @pl.when(wid >= _ACTIVE_W)
    def _zero():
        for c in range(_EMBED // 16):
            acc_v[0, pl.ds(c * 16, 16)] = jnp.zeros((16,), jnp.float32)

    pltpu.sync_copy(acc_v, out_hbm.at[pl.ds(wid, 1)])


@functools.cache
def _sc_gather():
    return pl.kernel(
        _sc_gather_sum,
        out_type=jax.ShapeDtypeStruct((_NW, _EMBED), jnp.float32),
        mesh=plsc.VectorSubcoreMesh(core_axis_name="c", subcore_axis_name="s"),
        scratch_types=[
            pltpu.VMEM((16,), jnp.int32),
            pltpu.VMEM((_IDX_PER_W, _EMBED), jnp.float32),
            pltpu.VMEM((1, _EMBED), jnp.float32),
            pltpu.SemaphoreType.DMA,
        ],
        compiler_params=pltpu.CompilerParams(needs_layout_passes=False),
    )


def _tc_logits(parts_ref, w1_ref, b1_ref, w2_ref, b2_ref, out_ref, hid_ref):
    i = pl.program_id(0)

    @pl.when(i == 0)
    def _head():
        x = jnp.sum(parts_ref[...], axis=0, keepdims=True)
        h = lax.dot_general(
            x, w1_ref[...], (((1,), (1,)), ((), ())),
            preferred_element_type=jnp.float32,
        ) + b1_ref[...]
        hid_ref[...] = jnp.maximum(h, 0.0)

    out_ref[...] = lax.dot_general(
        hid_ref[...], w2_ref[...], (((1,), (1,)), ((), ())),
        preferred_element_type=jnp.float32,
    ) + b2_ref[...].reshape(1, _TILE)


_tc_logits_call = pl.pallas_call(
    _tc_logits,
    grid=(_GRID,),
    in_specs=[
        pl.BlockSpec((_NW, _EMBED), lambda i: (0, 0)),
        pl.BlockSpec((_HIDDEN, _EMBED), lambda i: (0, 0)),
        pl.BlockSpec((1, _HIDDEN), lambda i: (0, 0)),
        pl.BlockSpec((_TILE, _HIDDEN), lambda i: (i, 0)),
        pl.BlockSpec((_TILE,), lambda i: (i,)),
    ],
    out_specs=pl.BlockSpec((1, _TILE), lambda i: (0, i)),
    out_shape=jax.ShapeDtypeStruct((1, _VOCAB), jnp.float32),
    scratch_shapes=[pltpu.VMEM((1, _HIDDEN), jnp.float32)],
    compiler_params=pltpu.CompilerParams(
        dimension_semantics=("arbitrary",),
    ),
)


def _tc_logsoftmax(lg_ref, out_ref):
    full = lg_ref[...]
    m = jnp.max(full)
    lse = m + jnp.log(jnp.sum(jnp.exp(full - m)))
    out_ref[...] = full - lse


_tc_norm_call = pl.pallas_call(
    _tc_logsoftmax,
    in_specs=[pl.BlockSpec((1, _VOCAB), lambda: (0, 0))],
    out_specs=pl.BlockSpec((1, _VOCAB), lambda: (0, 0)),
    out_shape=jax.ShapeDtypeStruct((1, _VOCAB), jnp.float32),
)


@jax.jit
def kernel(inputs, emb, W1, b1, W2, b2):
    parts = _sc_gather()(inputs, emb)
    logits = _tc_logits_call(parts, W1, b1.reshape(1, _HIDDEN), W2, b2)
    return _tc_norm_call(logits)


# speedup vs baseline: 2.1201x; 1.4588x over previous
import functools

import jax
import jax.numpy as jnp
from jax import lax
from jax.experimental import pallas as pl
from jax.experimental.pallas import tpu as pltpu
from jax.experimental.pallas import tpu_sc as plsc

_VOCAB = 100000
_EMBED = 64
_HIDDEN = 128
_CTX = 200

_NC = 2
_NS = 16
_NW = _NC * _NS
_IDX_PER_W = 8
_ACTIVE_W = _CTX // _IDX_PER_W

_TILE = 2048
_GRID = (_VOCAB + _TILE - 1) // _TILE
_EDGE = _VOCAB - (_GRID - 1) * _TILE


def _sc_gather_sum(idx_hbm, embt_hbm, out_hbm, idx_v, rows_v, acc_v, sem):
    wid = lax.axis_index("s") * _NC + lax.axis_index("c")

    @pl.when(wid < _ACTIVE_W)
    def _gather():
        base = pl.multiple_of(wid * _IDX_PER_W, _IDX_PER_W)
        pltpu.sync_copy(idx_hbm.at[pl.ds(base, _IDX_PER_W)],
                        idx_v.at[pl.ds(0, _IDX_PER_W)])
        iv = idx_v[...]
        lane = lax.iota(jnp.int32, 16)
        copies = []
        cols = []
        for j in range(_IDX_PER_W):
            col_j = jnp.sum(jnp.where(lane == j, iv, 0))
            start = pl.multiple_of((col_j >> 7) << 7, 128)
            cols.append(col_j & 127)
            copies.append(pltpu.make_async_copy(
                embt_hbm.at[:, pl.ds(start, 128)], rows_v.at[j], sem))
        for c in copies:
            c.start()
        for c in copies:
            c.wait()
        for c in range(_EMBED // 16):
            rid = lax.iota(jnp.int32, 16) + c * 16
            acc = jnp.zeros((16,), jnp.float32)
            for j in range(_IDX_PER_W):
                cj = jnp.broadcast_to(cols[j], (16,))
                acc = acc + plsc.load_gather(rows_v.at[j], [rid, cj])
            acc_v[0, pl.ds(c * 16, 16)] = acc

    @pl.when(wid >= _ACTIVE_W)
    def _zero():
        for c in range(_EMBED // 16):
            acc_v[0, pl.ds(c * 16, 16)] = jnp.zeros((16,), jnp.float32)

    pltpu.sync_copy(acc_v, out_hbm.at[pl.ds(wid, 1)])


@functools.cache
def _sc_gather():
    return pl.kernel(
        _sc_gather_sum,
        out_type=jax.ShapeDtypeStruct((_NW, _EMBED), jnp.float32),
        mesh=plsc.VectorSubcoreMesh(core_axis_name="c", subcore_axis_name="s"),
        scratch_types=[
            pltpu.VMEM((16,), jnp.int32),
            pltpu.VMEM((_IDX_PER_W, _EMBED, 128), jnp.float32),
            pltpu.VMEM((1, _EMBED), jnp.float32),
            pltpu.SemaphoreType.DMA,
        ],
        compiler_params=pltpu.CompilerParams(needs_layout_passes=False),
    )


def _tc_logits(parts_ref, w1_ref, b1_ref, w2_ref, b2_ref, out_ref, hid_ref):
    i = pl.program_id(0)

    @pl.when(i == 0)
    def _head():
        x = jnp.sum(parts_ref[...], axis=0, keepdims=True)
        h = lax.dot_general(
            x, w1_ref[...], (((1,), (1,)), ((), ())),
            preferred_element_type=jnp.float32,
        ) + b1_ref[...]
        hid_ref[...] = jnp.maximum(h, 0.0)

    out_ref[...] = lax.dot_general(
        hid_ref[...], w2_ref[...], (((1,), (1,)), ((), ())),
        preferred_element_type=jnp.float32,
    ) + b2_ref[...].reshape(1, _TILE)


_tc_logits_call = pl.pallas_call(
    _tc_logits,
    grid=(_GRID,),
    in_specs=[
        pl.BlockSpec((_NW, _EMBED), lambda i: (0, 0)),
        pl.BlockSpec((_HIDDEN, _EMBED), lambda i: (0, 0)),
        pl.BlockSpec((1, _HIDDEN), lambda i: (0, 0)),
        pl.BlockSpec((_TILE, _HIDDEN), lambda i: (i, 0)),
        pl.BlockSpec((_TILE,), lambda i: (i,)),
    ],
    out_specs=pl.BlockSpec((1, _TILE), lambda i: (0, i)),
    out_shape=jax.ShapeDtypeStruct((1, _VOCAB), jnp.float32),
    scratch_shapes=[pltpu.VMEM((1, _HIDDEN), jnp.float32)],
    compiler_params=pltpu.CompilerParams(
        dimension_semantics=("arbitrary",),
    ),
)


def _tc_logsoftmax(lg_ref, out_ref):
    full = lg_ref[...]
    m = jnp.max(full)
    lse = m + jnp.log(jnp.sum(jnp.exp(full - m)))
    out_ref[...] = full - lse


_tc_norm_call = pl.pallas_call(
    _tc_logsoftmax,
    in_specs=[pl.BlockSpec((1, _VOCAB), lambda: (0, 0))],
    out_specs=pl.BlockSpec((1, _VOCAB), lambda: (0, 0)),
    out_shape=jax.ShapeDtypeStruct((1, _VOCAB), jnp.float32),
)


@jax.jit
def kernel(inputs, emb, W1, b1, W2, b2):
    parts = _sc_gather()(inputs, emb.T)
    logits = _tc_logits_call(parts, W1, b1.reshape(1, _HIDDEN), W2, b2)
    return _tc_norm_call(logits)


# speedup vs baseline: 2.5769x; 1.2155x over previous
import functools

import jax
import jax.numpy as jnp
from jax import lax
from jax.experimental import pallas as pl
from jax.experimental.pallas import tpu as pltpu
from jax.experimental.pallas import tpu_sc as plsc

_VOCAB = 100000
_EMBED = 64
_HIDDEN = 128
_CTX = 200

_NC = 2
_NS = 16
_NW = _NC * _NS
_IDX_PER_W = 8
_ACTIVE_W = _CTX // _IDX_PER_W

_TILE = 4096
_GRID = (_VOCAB + _TILE - 1) // _TILE
_EDGE = _VOCAB - (_GRID - 1) * _TILE


def _sc_gather_sum(idx_hbm, embt_hbm, out_hbm, idx_v, rows_v, acc_v, sem):
    wid = lax.axis_index("s") * _NC + lax.axis_index("c")

    @pl.when(wid < _ACTIVE_W)
    def _gather():
        base = pl.multiple_of(wid * _IDX_PER_W, _IDX_PER_W)
        pltpu.sync_copy(idx_hbm.at[pl.ds(base, _IDX_PER_W)],
                        idx_v.at[pl.ds(0, _IDX_PER_W)])
        iv = idx_v[...]
        lane = lax.iota(jnp.int32, 16)
        copies = []
        cols = []
        for j in range(_IDX_PER_W):
            col_j = jnp.sum(jnp.where(lane == j, iv, 0))
            start = pl.multiple_of((col_j >> 7) << 7, 128)
            cols.append(col_j & 127)
            copies.append(pltpu.make_async_copy(
                embt_hbm.at[:, pl.ds(start, 128)], rows_v.at[j], sem))
        for c in copies:
            c.start()
        for c in copies:
            c.wait()
        for c in range(_EMBED // 16):
            rid = lax.iota(jnp.int32, 16) + c * 16
            acc = jnp.zeros((16,), jnp.float32)
            for j in range(_IDX_PER_W):
                cj = jnp.broadcast_to(cols[j], (16,))
                acc = acc + plsc.load_gather(rows_v.at[j], [rid, cj])
            acc_v[0, pl.ds(c * 16, 16)] = acc

    @pl.when(wid >= _ACTIVE_W)
    def _zero():
        for c in range(_EMBED // 16):
            acc_v[0, pl.ds(c * 16, 16)] = jnp.zeros((16,), jnp.float32)

    pltpu.sync_copy(acc_v, out_hbm.at[pl.ds(wid, 1)])


@functools.cache
def _sc_gather():
    return pl.kernel(
        _sc_gather_sum,
        out_type=jax.ShapeDtypeStruct((_NW, _EMBED), jnp.float32),
        mesh=plsc.VectorSubcoreMesh(core_axis_name="c", subcore_axis_name="s"),
        scratch_types=[
            pltpu.VMEM((16,), jnp.int32),
            pltpu.VMEM((_IDX_PER_W, _EMBED, 128), jnp.float32),
            pltpu.VMEM((1, _EMBED), jnp.float32),
            pltpu.SemaphoreType.DMA,
        ],
        compiler_params=pltpu.CompilerParams(needs_layout_passes=False),
    )


def _tc_logits(parts_ref, w1_ref, b1_ref, w2_ref, b2_ref, out_ref, hid_ref):
    i = pl.program_id(0)

    @pl.when(i == 0)
    def _head():
        x = jnp.sum(parts_ref[...], axis=0, keepdims=True)
        h = lax.dot_general(
            x, w1_ref[...], (((1,), (1,)), ((), ())),
            preferred_element_type=jnp.float32,
        ) + b1_ref[...]
        hid_ref[...] = jnp.maximum(h, 0.0)

    out_ref[...] = lax.dot_general(
        hid_ref[...], w2_ref[...], (((1,), (1,)), ((), ())),
        preferred_element_type=jnp.float32,
    ) + b2_ref[...].reshape(1, _TILE)


_tc_logits_call = pl.pallas_call(
    _tc_logits,
    grid=(_GRID,),
    in_specs=[
        pl.BlockSpec((_NW, _EMBED), lambda i: (0, 0)),
        pl.BlockSpec((_HIDDEN, _EMBED), lambda i: (0, 0)),
        pl.BlockSpec((1, _HIDDEN), lambda i: (0, 0)),
        pl.BlockSpec((_TILE, _HIDDEN), lambda i: (i, 0)),
        pl.BlockSpec((_TILE,), lambda i: (i,)),
    ],
    out_specs=pl.BlockSpec((1, _TILE), lambda i: (0, i)),
    out_shape=jax.ShapeDtypeStruct((1, _VOCAB), jnp.float32),
    scratch_shapes=[pltpu.VMEM((1, _HIDDEN), jnp.float32)],
    compiler_params=pltpu.CompilerParams(
        dimension_semantics=("arbitrary",),
    ),
)


def _tc_logsoftmax(lg_ref, out_ref):
    full = lg_ref[...]
    m = jnp.max(full)
    lse = m + jnp.log(jnp.sum(jnp.exp(full - m)))
    out_ref[...] = full - lse


_tc_norm_call = pl.pallas_call(
    _tc_logsoftmax,
    in_specs=[pl.BlockSpec((1, _VOCAB), lambda: (0, 0))],
    out_specs=pl.BlockSpec((1, _VOCAB), lambda: (0, 0)),
    out_shape=jax.ShapeDtypeStruct((1, _VOCAB), jnp.float32),
)


@jax.jit
def kernel(inputs, emb, W1, b1, W2, b2):
    parts = _sc_gather()(inputs, emb.T)
    logits = _tc_logits_call(parts, W1, b1.reshape(1, _HIDDEN), W2, b2)
    return _tc_norm_call(logits)


# speedup vs baseline: 2.9649x; 1.1506x over previous
import functools

import jax
import jax.numpy as jnp
from jax import lax
from jax.experimental import pallas as pl
from jax.experimental.pallas import tpu as pltpu
from jax.experimental.pallas import tpu_sc as plsc

_VOCAB = 100000
_EMBED = 64
_HIDDEN = 128
_CTX = 200

_NC = 2
_NS = 16
_NW = _NC * _NS
_IDX_PER_W = 8
_ACTIVE_W = _CTX // _IDX_PER_W

_TILE = 8192
_GRID = (_VOCAB + _TILE - 1) // _TILE
_EDGE = _VOCAB - (_GRID - 1) * _TILE


def _sc_gather_sum(idx_hbm, embt_hbm, out_hbm, idx_v, rows_v, acc_v, sem):
    wid = lax.axis_index("s") * _NC + lax.axis_index("c")

    @pl.when(wid < _ACTIVE_W)
    def _gather():
        base = pl.multiple_of(wid * _IDX_PER_W, _IDX_PER_W)
        pltpu.sync_copy(idx_hbm.at[pl.ds(base, _IDX_PER_W)],
                        idx_v.at[pl.ds(0, _IDX_PER_W)])
        iv = idx_v[...]
        lane = lax.iota(jnp.int32, 16)
        copies = []
        cols = []
        for j in range(_IDX_PER_W):
            col_j = jnp.sum(jnp.where(lane == j, iv, 0))
            start = pl.multiple_of((col_j >> 7) << 7, 128)
            cols.append(col_j & 127)
            copies.append(pltpu.make_async_copy(
                embt_hbm.at[:, pl.ds(start, 128)], rows_v.at[j], sem))
        for c in copies:
            c.start()
        for c in copies:
            c.wait()
        for c in range(_EMBED // 16):
            rid = lax.iota(jnp.int32, 16) + c * 16
            acc = jnp.zeros((16,), jnp.float32)
            for j in range(_IDX_PER_W):
                cj = jnp.broadcast_to(cols[j], (16,))
                acc = acc + plsc.load_gather(rows_v.at[j], [rid, cj])
            acc_v[0, pl.ds(c * 16, 16)] = acc

    @pl.when(wid >= _ACTIVE_W)
    def _zero():
        for c in range(_EMBED // 16):
            acc_v[0, pl.ds(c * 16, 16)] = jnp.zeros((16,), jnp.float32)

    pltpu.sync_copy(acc_v, out_hbm.at[pl.ds(wid, 1)])


@functools.cache
def _sc_gather():
    return pl.kernel(
        _sc_gather_sum,
        out_type=jax.ShapeDtypeStruct((_NW, _EMBED), jnp.float32),
        mesh=plsc.VectorSubcoreMesh(core_axis_name="c", subcore_axis_name="s"),
        scratch_types=[
            pltpu.VMEM((16,), jnp.int32),
            pltpu.VMEM((_IDX_PER_W, _EMBED, 128), jnp.float32),
            pltpu.VMEM((1, _EMBED), jnp.float32),
            pltpu.SemaphoreType.DMA,
        ],
        compiler_params=pltpu.CompilerParams(needs_layout_passes=False),
    )


def _tc_logits(parts_ref, w1_ref, b1_ref, w2_ref, b2_ref, out_ref, hid_ref):
    i = pl.program_id(0)

    @pl.when(i == 0)
    def _head():
        x = jnp.sum(parts_ref[...], axis=0, keepdims=True)
        h = lax.dot_general(
            x, w1_ref[...], (((1,), (1,)), ((), ())),
            preferred_element_type=jnp.float32,
        ) + b1_ref[...]
        hid_ref[...] = jnp.maximum(h, 0.0)

    out_ref[...] = lax.dot_general(
        hid_ref[...], w2_ref[...], (((1,), (1,)), ((), ())),
        preferred_element_type=jnp.float32,
    ) + b2_ref[...].reshape(1, _TILE)


_tc_logits_call = pl.pallas_call(
    _tc_logits,
    grid=(_GRID,),
    in_specs=[
        pl.BlockSpec((_NW, _EMBED), lambda i: (0, 0)),
        pl.BlockSpec((_HIDDEN, _EMBED), lambda i: (0, 0)),
        pl.BlockSpec((1, _HIDDEN), lambda i: (0, 0)),
        pl.BlockSpec((_TILE, _HIDDEN), lambda i: (i, 0)),
        pl.BlockSpec((_TILE,), lambda i: (i,)),
    ],
    out_specs=pl.BlockSpec((1, _TILE), lambda i: (0, i)),
    out_shape=jax.ShapeDtypeStruct((1, _VOCAB), jnp.float32),
    scratch_shapes=[pltpu.VMEM((1, _HIDDEN), jnp.float32)],
    compiler_params=pltpu.CompilerParams(
        dimension_semantics=("arbitrary",),
    ),
)


def _tc_logsoftmax(lg_ref, out_ref):
    full = lg_ref[...]
    m = jnp.max(full)
    lse = m + jnp.log(jnp.sum(jnp.exp(full - m)))
    out_ref[...] = full - lse


_tc_norm_call = pl.pallas_call(
    _tc_logsoftmax,
    in_specs=[pl.BlockSpec((1, _VOCAB), lambda: (0, 0))],
    out_specs=pl.BlockSpec((1, _VOCAB), lambda: (0, 0)),
    out_shape=jax.ShapeDtypeStruct((1, _VOCAB), jnp.float32),
)


@jax.jit
def kernel(inputs, emb, W1, b1, W2, b2):
    parts = _sc_gather()(inputs, emb.T)
    logits = _tc_logits_call(parts, W1, b1.reshape(1, _HIDDEN), W2, b2)
    return _tc_norm_call(logits)


# speedup vs baseline: 3.0825x; 1.0397x over previous
import functools

import jax
import jax.numpy as jnp
from jax import lax
from jax.experimental import pallas as pl
from jax.experimental.pallas import tpu as pltpu
from jax.experimental.pallas import tpu_sc as plsc

_VOCAB = 100000
_EMBED = 64
_HIDDEN = 128
_CTX = 200

_NC = 2
_NS = 16
_NW = _NC * _NS
_IDX_PER_W = 8
_ACTIVE_W = _CTX // _IDX_PER_W

_TILE = 16384
_GRID = (_VOCAB + _TILE - 1) // _TILE
_EDGE = _VOCAB - (_GRID - 1) * _TILE


def _sc_gather_sum(idx_hbm, embt_hbm, out_hbm, idx_v, rows_v, acc_v, sem):
    wid = lax.axis_index("s") * _NC + lax.axis_index("c")

    @pl.when(wid < _ACTIVE_W)
    def _gather():
        base = pl.multiple_of(wid * _IDX_PER_W, _IDX_PER_W)
        pltpu.sync_copy(idx_hbm.at[pl.ds(base, _IDX_PER_W)],
                        idx_v.at[pl.ds(0, _IDX_PER_W)])
        iv = idx_v[...]
        lane = lax.iota(jnp.int32, 16)
        copies = []
        cols = []
        for j in range(_IDX_PER_W):
            col_j = jnp.sum(jnp.where(lane == j, iv, 0))
            start = pl.multiple_of((col_j >> 7) << 7, 128)
            cols.append(col_j & 127)
            copies.append(pltpu.make_async_copy(
                embt_hbm.at[:, pl.ds(start, 128)], rows_v.at[j], sem))
        for c in copies:
            c.start()
        for c in copies:
            c.wait()
        for c in range(_EMBED // 16):
            rid = lax.iota(jnp.int32, 16) + c * 16
            acc = jnp.zeros((16,), jnp.float32)
            for j in range(_IDX_PER_W):
                cj = jnp.broadcast_to(cols[j], (16,))
                acc = acc + plsc.load_gather(rows_v.at[j], [rid, cj])
            acc_v[0, pl.ds(c * 16, 16)] = acc

    @pl.when(wid >= _ACTIVE_W)
    def _zero():
        for c in range(_EMBED // 16):
            acc_v[0, pl.ds(c * 16, 16)] = jnp.zeros((16,), jnp.float32)

    pltpu.sync_copy(acc_v, out_hbm.at[pl.ds(wid, 1)])


@functools.cache
def _sc_gather():
    return pl.kernel(
        _sc_gather_sum,
        out_type=jax.ShapeDtypeStruct((_NW, _EMBED), jnp.float32),
        mesh=plsc.VectorSubcoreMesh(core_axis_name="c", subcore_axis_name="s"),
        scratch_types=[
            pltpu.VMEM((16,), jnp.int32),
            pltpu.VMEM((_IDX_PER_W, _EMBED, 128), jnp.float32),
            pltpu.VMEM((1, _EMBED), jnp.float32),
            pltpu.SemaphoreType.DMA,
        ],
        compiler_params=pltpu.CompilerParams(needs_layout_passes=False),
    )


def _tc_logits(parts_ref, w1_ref, b1_ref, w2_ref, b2_ref, out_ref, hid_ref):
    i = pl.program_id(0)

    @pl.when(i == 0)
    def _head():
        x = jnp.sum(parts_ref[...], axis=0, keepdims=True)
        h = lax.dot_general(
            x, w1_ref[...], (((1,), (1,)), ((), ())),
            preferred_element_type=jnp.float32,
        ) + b1_ref[...]
        hid_ref[...] = jnp.maximum(h, 0.0)

    out_ref[...] = lax.dot_general(
        hid_ref[...], w2_ref[...], (((1,), (1,)), ((), ())),
        preferred_element_type=jnp.float32,
    ) + b2_ref[...].reshape(1, _TILE)


_tc_logits_call = pl.pallas_call(
    _tc_logits,
    grid=(_GRID,),
    in_specs=[
        pl.BlockSpec((_NW, _EMBED), lambda i: (0, 0)),
        pl.BlockSpec((_HIDDEN, _EMBED), lambda i: (0, 0)),
        pl.BlockSpec((1, _HIDDEN), lambda i: (0, 0)),
        pl.BlockSpec((_TILE, _HIDDEN), lambda i: (i, 0)),
        pl.BlockSpec((_TILE,), lambda i: (i,)),
    ],
    out_specs=pl.BlockSpec((1, _TILE), lambda i: (0, i)),
    out_shape=jax.ShapeDtypeStruct((1, _VOCAB), jnp.float32),
    scratch_shapes=[pltpu.VMEM((1, _HIDDEN), jnp.float32)],
    compiler_params=pltpu.CompilerParams(
        dimension_semantics=("arbitrary",),
    ),
)


def _tc_logsoftmax(lg_ref, out_ref):
    full = lg_ref[...]
    m = jnp.max(full)
    lse = m + jnp.log(jnp.sum(jnp.exp(full - m)))
    out_ref[...] = full - lse


_tc_norm_call = pl.pallas_call(
    _tc_logsoftmax,
    in_specs=[pl.BlockSpec((1, _VOCAB), lambda: (0, 0))],
    out_specs=pl.BlockSpec((1, _VOCAB), lambda: (0, 0)),
    out_shape=jax.ShapeDtypeStruct((1, _VOCAB), jnp.float32),
)


@jax.jit
def kernel(inputs, emb, W1, b1, W2, b2):
    parts = _sc_gather()(inputs, emb.T)
    logits = _tc_logits_call(parts, W1, b1.reshape(1, _HIDDEN), W2, b2)
    return _tc_norm_call(logits)
